# Initial kernel scaffold; baseline (speedup 1.0000x reference)
import jax
import jax.numpy as jnp
from jax.experimental import pallas as pl


def kernel(ego_embeddings, adj_rows, adj_cols, adj_vals, W):
    raise NotImplementedError("write your pallas kernel here")


# speedup vs baseline: 3.7593x; 3.7593x over previous
import functools

import jax
import jax.numpy as jnp
from jax import lax
from jax.experimental import pallas as pl
from jax.experimental.pallas import tpu as pltpu
from jax.experimental.pallas import tpu_sc as plsc

_N = 10000
_D = 128
_NW = 32
_CHUNK = 128
_NCHUNK = 80
_EPAD = _NW * _NCHUNK * _CHUNK
_NPAD = 10240
_STRIPE = _NPAD // 16


def _sc_spmm(ego, cols3, rows3, vals3):
    mesh = plsc.VectorSubcoreMesh(core_axis_name="c", subcore_axis_name="s")

    @functools.partial(
        pl.kernel,
        mesh=mesh,
        out_type=jax.ShapeDtypeStruct((2, _NPAD, _D), jnp.float32),
        scratch_types=[
            pltpu.VMEM((_NCHUNK, _CHUNK), jnp.int32),
            pltpu.VMEM((_NCHUNK, _CHUNK), jnp.int32),
            pltpu.VMEM((_NCHUNK, _CHUNK), jnp.float32),
            pltpu.VMEM((_CHUNK, _D), jnp.float32),
            pltpu.VMEM_SHARED((_NPAD, _D), jnp.float32),
            pltpu.SemaphoreType.DMA,
        ],
    )
    def k(ego_hbm, cols_hbm, rows_hbm, vals_hbm, out_hbm,
          cols_v, rows_v, vals_v, gbuf, acc, sem):
        c = lax.axis_index("c")
        s = lax.axis_index("s")
        wid = s * 2 + c

        pltpu.sync_copy(cols_hbm.at[wid], cols_v)
        pltpu.sync_copy(rows_hbm.at[wid], rows_v)
        pltpu.sync_copy(vals_hbm.at[wid], vals_v)

        zv = jnp.zeros((16,), jnp.float32)

        def zrow(i, carry):
            for f in range(8):
                gbuf[i, pl.ds(16 * f, 16)] = zv
            return carry

        lax.fori_loop(0, _CHUNK, zrow, 0)
        for p in range(_STRIPE // _CHUNK):
            pltpu.sync_copy(gbuf, acc.at[pl.ds(s * _STRIPE + p * _CHUNK, _CHUNK)])
        plsc.subcore_barrier()

        def chunk_body(j, carry):
            pltpu.async_copy(ego_hbm.at[cols_v.at[j]], gbuf, sem).wait()

            def egroup(g, icarry):
                vv = vals_v[j, pl.ds(16 * g, 16)]
                base = 16 * g
                for i2 in range(16):
                    v = vv[i2]
                    for f in range(8):
                        sl = pl.ds(16 * f, 16)
                        gbuf[base + i2, sl] = gbuf[base + i2, sl] * v
                return icarry

            lax.fori_loop(0, _CHUNK // 16, egroup, 0)
            pltpu.sync_copy(gbuf, acc.at[rows_v.at[j]], add=True)
            return carry

        lax.fori_loop(0, _NCHUNK, chunk_body, 0)
        plsc.subcore_barrier()

        pltpu.sync_copy(acc.at[pl.ds(s * _STRIPE, _STRIPE)],
                        out_hbm.at[c, pl.ds(s * _STRIPE, _STRIPE)])

    return k(ego, cols3, rows3, vals3)


def _tc_finish(p0, p1, w_t):
    blk = 2000

    def mm(p0_ref, p1_ref, w_ref, o_ref):
        x = p0_ref[...] + p1_ref[...]
        o_ref[...] = jnp.dot(x, w_ref[...], preferred_element_type=jnp.float32)

    return pl.pallas_call(
        mm,
        grid=(_N // blk,),
        in_specs=[
            pl.BlockSpec((blk, _D), lambda i: (i, 0)),
            pl.BlockSpec((blk, _D), lambda i: (i, 0)),
            pl.BlockSpec((_D, _D), lambda i: (0, 0)),
        ],
        out_specs=pl.BlockSpec((blk, _D), lambda i: (i, 0)),
        out_shape=jax.ShapeDtypeStruct((_N, _D), jnp.float32),
    )(p0, p1, w_t)


def kernel(ego_embeddings, adj_rows, adj_cols, adj_vals, W):
    e = adj_rows.shape[0]
    pad = _EPAD - e
    cols3 = jnp.pad(adj_cols.astype(jnp.int32), (0, pad)).reshape(_NW, _NCHUNK, _CHUNK)
    rows3 = jnp.pad(adj_rows.astype(jnp.int32), (0, pad)).reshape(_NW, _NCHUNK, _CHUNK)
    vals3 = jnp.pad(adj_vals, (0, pad)).reshape(_NW, _NCHUNK, _CHUNK)
    parts = _sc_spmm(ego_embeddings, cols3, rows3, vals3)
    return _tc_finish(parts[0, :_N], parts[1, :_N], W.T)


# speedup vs baseline: 4.6407x; 1.2345x over previous
import functools

import jax
import jax.numpy as jnp
from jax import lax
from jax.experimental import pallas as pl
from jax.experimental.pallas import tpu as pltpu
from jax.experimental.pallas import tpu_sc as plsc

_N = 10000
_D = 128
_NW = 32
_CHUNK = 80
_NCHUNK = 128
_EPAD = _NW * _NCHUNK * _CHUNK
_NPAD = 10240
_STRIPE = _NPAD // 16
_NB = 4
_GRP = _CHUNK // 16


def _sc_spmm(ego, edges3, vals3):
    mesh = plsc.VectorSubcoreMesh(core_axis_name="c", subcore_axis_name="s")

    @functools.partial(
        pl.kernel,
        mesh=mesh,
        out_type=jax.ShapeDtypeStruct((2, _NPAD, _D), jnp.float32),
        scratch_types=[
            pltpu.VMEM((_NB, 2, _CHUNK), jnp.int32),
            pltpu.VMEM((_NB, _CHUNK), jnp.float32),
            pltpu.VMEM((_NB, _CHUNK), jnp.int32),
            pltpu.VMEM((_NB, _CHUNK, _D), jnp.float32),
            pltpu.VMEM_SHARED((_NPAD, _D), jnp.float32),
        ]
        + [pltpu.SemaphoreType.DMA] * (4 * _NB),
    )
    def k(ego_hbm, edges_hbm, vals_hbm, out_hbm, ebuf, vbuf, ibuf, gbuf,
          acc, *sems):
        esems = sems[0:_NB]
        vsems = sems[_NB:2 * _NB]
        gsems = sems[2 * _NB:3 * _NB]
        ssems = sems[3 * _NB:4 * _NB]
        c = lax.axis_index("c")
        s = lax.axis_index("s")
        wid = s * 2 + c

        def edge_copy(j, d):
            return pltpu.make_async_copy(
                edges_hbm.at[wid, j], ebuf.at[d], esems[d])

        def val_copy(j, d):
            return pltpu.make_async_copy(
                vals_hbm.at[wid, j], vbuf.at[d], vsems[d])

        def gather_copy(b):
            return pltpu.make_async_copy(
                ego_hbm.at[ebuf.at[b, 0]], gbuf.at[b], gsems[b])

        def scatter_copy(b):
            return pltpu.make_async_copy(
                gbuf.at[b], acc.at[ibuf.at[b]], ssems[b])

        def scale(b):
            def group(g, carry):
                sl = pl.ds(16 * g, 16)
                ibuf[b, sl] = ebuf[b, 1, sl]
                vv = vbuf[b, sl]
                base = 16 * g
                for i2 in range(16):
                    v = vv[i2]
                    for f in range(8):
                        fs = pl.ds(16 * f, 16)
                        gbuf[b, base + i2, fs] = gbuf[b, base + i2, fs] * v
                return carry
            lax.fori_loop(0, _GRP, group, 0)

        zv = jnp.zeros((16,), jnp.float32)

        def zrow(i, carry):
            for f in range(8):
                gbuf[0, i, pl.ds(16 * f, 16)] = zv
            return carry

        lax.fori_loop(0, _CHUNK, zrow, 0)
        for p in range(_STRIPE // _CHUNK):
            pltpu.sync_copy(gbuf.at[0],
                            acc.at[pl.ds(s * _STRIPE + p * _CHUNK, _CHUNK)])
        plsc.subcore_barrier()

        for t in range(3):
            edge_copy(t, t).start()
            val_copy(t, t).start()
        edge_copy(0, 0).wait()
        val_copy(0, 0).wait()
        gather_copy(0).start()
        edge_copy(1, 1).wait()
        val_copy(1, 1).wait()
        gather_copy(1).start()

        def step(j, b, scatter_wait, edge_start, gather_start):
            gather_copy(b).wait()
            scale(b)
            scatter_copy(b).start(add=True)
            if edge_start:
                edge_copy(j + 3, (b + 3) % _NB).start()
                val_copy(j + 3, (b + 3) % _NB).start()
            if scatter_wait:
                scatter_copy((b + 2) % _NB).wait()
            if gather_start:
                edge_copy(j + 2, (b + 2) % _NB).wait()
                val_copy(j + 2, (b + 2) % _NB).wait()
                gather_copy((b + 2) % _NB).start()

        for b in range(_NB):
            step(b, b, scatter_wait=(b >= 2), edge_start=True,
                 gather_start=True)

        def main(jj, carry):
            j0 = jj * _NB
            for b in range(_NB):
                step(j0 + b, b, scatter_wait=True, edge_start=True,
                     gather_start=True)
            return carry

        lax.fori_loop(1, (_NCHUNK - _NB) // _NB, main, 0)

        for b in range(_NB):
            j = _NCHUNK - _NB + b
            step(j, b, scatter_wait=True, edge_start=(j + 3 < _NCHUNK),
                 gather_start=(j + 2 < _NCHUNK))
        scatter_copy((_NCHUNK - 2) % _NB).wait()
        scatter_copy((_NCHUNK - 1) % _NB).wait()

        plsc.subcore_barrier()

        pltpu.sync_copy(acc.at[pl.ds(s * _STRIPE, _STRIPE)],
                        out_hbm.at[c, pl.ds(s * _STRIPE, _STRIPE)])

    return k(ego, edges3, vals3)


def _tc_finish(p0, p1, w_t):
    blk = 2000

    def mm(p0_ref, p1_ref, w_ref, o_ref):
        x = p0_ref[...] + p1_ref[...]
        o_ref[...] = jnp.dot(x, w_ref[...], preferred_element_type=jnp.float32)

    return pl.pallas_call(
        mm,
        grid=(_N // blk,),
        in_specs=[
            pl.BlockSpec((blk, _D), lambda i: (i, 0)),
            pl.BlockSpec((blk, _D), lambda i: (i, 0)),
            pl.BlockSpec((_D, _D), lambda i: (0, 0)),
        ],
        out_specs=pl.BlockSpec((blk, _D), lambda i: (i, 0)),
        out_shape=jax.ShapeDtypeStruct((_N, _D), jnp.float32),
    )(p0, p1, w_t)


def kernel(ego_embeddings, adj_rows, adj_cols, adj_vals, W):
    e = adj_rows.shape[0]
    pad = _EPAD - e
    cols3 = jnp.pad(adj_cols.astype(jnp.int32), (0, pad)).reshape(
        _NW, _NCHUNK, _CHUNK)
    rows3 = jnp.pad(adj_rows.astype(jnp.int32), (0, pad)).reshape(
        _NW, _NCHUNK, _CHUNK)
    vals3 = jnp.pad(adj_vals, (0, pad)).reshape(_NW, _NCHUNK, _CHUNK)
    edges3 = jnp.stack([cols3, rows3], axis=2)
    parts = _sc_spmm(ego_embeddings, edges3, vals3)
    return _tc_finish(parts[0, :_N], parts[1, :_N], W.T)
